# Initial kernel scaffold; baseline (speedup 1.0000x reference)
import jax
import jax.numpy as jnp
from jax.experimental import pallas as pl


def kernel(x, base, ln_d_g, ln_d_b, W1, b1, W2, b2, ln_u_g, ln_u_b, U1, c1, U2, c2, indices):
    raise NotImplementedError("write your pallas kernel here")


# speedup vs baseline: 2.0906x; 2.0906x over previous
import functools
import math

import jax
import jax.numpy as jnp
from jax.experimental import pallas as pl
from jax.experimental.pallas import tpu as pltpu

T, A, N, D, C = 4, 50000, 100000, 64, 64

_SQRT_HALF = 0.7071067811865476


def _gelu_exact(v):
    return 0.5 * v * (1.0 + jax.lax.erf(v * _SQRT_HALF))


def _ln(v, g, b, eps=1e-5):
    mu = jnp.mean(v, axis=-1, keepdims=True)
    var = jnp.mean((v - mu) ** 2, axis=-1, keepdims=True)
    return (v - mu) * jax.lax.rsqrt(var + eps) * g + b


def _mlp_body(y_ref, gd_ref, bd_ref, W1_ref, b1_ref, W2_ref, b2_ref,
              gu_ref, bu_ref, U1_ref, c1_ref, U2_ref, c2_ref, out_ref):
    y = y_ref[...]
    h = _ln(y, gd_ref[...], bd_ref[...])
    h = _gelu_exact(jnp.dot(h, W1_ref[...], preferred_element_type=jnp.float32)
                    + b1_ref[...])
    e = jnp.dot(h, W2_ref[...], preferred_element_type=jnp.float32) + b2_ref[...]
    g = _ln(e, gu_ref[...], bu_ref[...])
    g = _gelu_exact(jnp.dot(g, U1_ref[...], preferred_element_type=jnp.float32)
                    + c1_ref[...])
    out_ref[...] = (jnp.dot(g, U2_ref[...], preferred_element_type=jnp.float32)
                    + c2_ref[...])


def _fused_mlp(y, ln_d_g, ln_d_b, W1, b1, W2, b2, ln_u_g, ln_u_b, U1, c1, U2, c2,
               interpret=False):
    R = y.shape[0]
    BR = 2000
    grid = (R // BR,)
    full = lambda shape: pl.BlockSpec(shape, lambda i: (0, 0))
    return pl.pallas_call(
        _mlp_body,
        grid=grid,
        in_specs=[
            pl.BlockSpec((BR, D), lambda i: (i, 0)),
            full((1, D)), full((1, D)),
            full((D, 2 * D)), full((1, 2 * D)),
            full((2 * D, C)), full((1, C)),
            full((1, C)), full((1, C)),
            full((C, 2 * C)), full((1, 2 * C)),
            full((2 * C, D)), full((1, D)),
        ],
        out_specs=pl.BlockSpec((BR, D), lambda i: (i, 0)),
        out_shape=jax.ShapeDtypeStruct((R, D), jnp.float32),
        interpret=interpret,
    )(y, ln_d_g.reshape(1, D), ln_d_b.reshape(1, D), W1, b1.reshape(1, 2 * D),
      W2, b2.reshape(1, C), ln_u_g.reshape(1, C), ln_u_b.reshape(1, C),
      U1, c1.reshape(1, 2 * C), U2, c2.reshape(1, D))


def kernel(x, base, ln_d_g, ln_d_b, W1, b1, W2, b2, ln_u_g, ln_u_b, U1, c1, U2, c2, indices):
    idx = indices.astype(jnp.int32)
    acc = jax.vmap(lambda b, i, xt: b.at[i].add(xt))(base, idx, x)
    y = jax.vmap(lambda a, i: a[i])(acc, idx).reshape(T * A, D)
    return _fused_mlp(y, ln_d_g, ln_d_b, W1, b1, W2, b2,
                      ln_u_g, ln_u_b, U1, c1, U2, c2)
